# Initial kernel scaffold; baseline (speedup 1.0000x reference)
import jax
import jax.numpy as jnp
from jax.experimental import pallas as pl


def kernel(h, edge_index, W0, b0, W1, b1, W2, b2):
    raise NotImplementedError("write your pallas kernel here")


# speedup vs baseline: 6.9406x; 6.9406x over previous
import jax
import jax.numpy as jnp
from jax import lax
from jax.experimental import pallas as pl
from jax.experimental.pallas import tpu as pltpu
from jax.experimental.pallas import tpu_sc as plsc

N, E, D = 10000, 320000, 128
NC, NS = 2, 16
NW = NC * NS
EPT = E // NW
CH = 80
NCH = EPT // CH
DEGW = 16
NP = 10240
RPS = NP // NS
ZR = 128
LANES = 16

_SC_PARAMS = pltpu.CompilerParams(use_tc_tiling_on_sc=False)


def _zero_rows(ref, nrows, ncols):
    z16 = jnp.zeros((LANES,), jnp.float32)

    def row(r, carry):
        for c in range(ncols // LANES):
            ref[r, pl.ds(c * LANES, LANES)] = z16
        return carry

    lax.fori_loop(0, nrows, row, 0)


def _worker_id():
    return lax.axis_index("s") * NC + lax.axis_index("c")


def _sc_agg_body(h_hbm, src_hbm, dst_hbm, agg_hbm,
                 src_v, dst_v, rows_v, z_v, acc_sh, sem):
    cid = lax.axis_index("c")
    sid = lax.axis_index("s")
    wid = _worker_id()

    pltpu.sync_copy(src_hbm.at[wid], src_v)
    pltpu.sync_copy(dst_hbm.at[wid], dst_v)

    _zero_rows(z_v, ZR, D)
    base = sid * RPS
    for k in range(RPS // ZR):
        pltpu.sync_copy(z_v, acc_sh.at[pl.ds(base + k * ZR, ZR)])

    plsc.subcore_barrier()

    def chunk(c, carry):
        pltpu.async_copy(h_hbm.at[src_v.at[c]], rows_v, sem).wait()
        pltpu.sync_copy(rows_v, acc_sh.at[dst_v.at[c]], add=True)
        return carry

    lax.fori_loop(0, NCH, chunk, 0)

    plsc.subcore_barrier()

    pltpu.sync_copy(acc_sh.at[pl.ds(base, RPS)],
                    agg_hbm.at[cid, pl.ds(base, RPS)])


_sc_agg = pl.kernel(
    _sc_agg_body,
    out_type=jax.ShapeDtypeStruct((NC, NP, D), jnp.float32),
    mesh=plsc.VectorSubcoreMesh(core_axis_name="c", subcore_axis_name="s"),
    scratch_types=(
        pltpu.VMEM((NCH, CH), jnp.int32),
        pltpu.VMEM((NCH, CH), jnp.int32),
        pltpu.VMEM((CH, D), jnp.float32),
        pltpu.VMEM((ZR, D), jnp.float32),
        pltpu.VMEM_SHARED((NP, D), jnp.float32),
        pltpu.SemaphoreType.DMA,
    ),
    compiler_params=_SC_PARAMS,
)


def _sc_deg_body(dst_hbm, deg_hbm, dst_v, zd_v, ones_v, dacc_sh):
    cid = lax.axis_index("c")
    sid = lax.axis_index("s")
    wid = _worker_id()

    pltpu.sync_copy(dst_hbm.at[wid], dst_v)

    _zero_rows(zd_v, RPS, DEGW)
    base = sid * RPS
    pltpu.sync_copy(zd_v, dacc_sh.at[pl.ds(base, RPS)])

    one16 = jnp.ones((LANES,), jnp.float32)

    def orow(r, carry):
        ones_v[r, pl.ds(0, LANES)] = one16
        return carry

    lax.fori_loop(0, CH, orow, 0)

    plsc.subcore_barrier()

    def chunk(c, carry):
        pltpu.sync_copy(ones_v, dacc_sh.at[dst_v.at[c]], add=True)
        return carry

    lax.fori_loop(0, NCH, chunk, 0)

    plsc.subcore_barrier()

    pltpu.sync_copy(dacc_sh.at[pl.ds(base, RPS)],
                    deg_hbm.at[cid, pl.ds(base, RPS)])


_sc_deg = pl.kernel(
    _sc_deg_body,
    out_type=jax.ShapeDtypeStruct((NC, NP, DEGW), jnp.float32),
    mesh=plsc.VectorSubcoreMesh(core_axis_name="c", subcore_axis_name="s"),
    scratch_types=(
        pltpu.VMEM((NCH, CH), jnp.int32),
        pltpu.VMEM((RPS, DEGW), jnp.float32),
        pltpu.VMEM((CH, DEGW), jnp.float32),
        pltpu.VMEM_SHARED((NP, DEGW), jnp.float32),
    ),
    compiler_params=_SC_PARAMS,
)

BN = 1000


def _dense0_body(aggp_ref, degp_ref, w_ref, b_ref, h_ref, degb_ref):
    p = aggp_ref[...]
    agg = p[0] + p[1]
    d = degp_ref[...]
    deg = (jnp.sum(d[0], axis=1) + jnp.sum(d[1], axis=1)) * (1.0 / DEGW)
    deg = jnp.maximum(deg, 1.0)[:, None]
    x = agg / deg
    y = jnp.dot(x, w_ref[...], preferred_element_type=jnp.float32)
    h_ref[...] = jnp.maximum(y + b_ref[...], 0.0)
    degb_ref[...] = jnp.broadcast_to(deg, (BN, D))


def _dense0(aggp, degp, w, b):
    return pl.pallas_call(
        _dense0_body,
        grid=(N // BN,),
        in_specs=[
            pl.BlockSpec((NC, BN, D), lambda i: (0, i, 0)),
            pl.BlockSpec((NC, BN, DEGW), lambda i: (0, i, 0)),
            pl.BlockSpec((D, D), lambda i: (0, 0)),
            pl.BlockSpec((1, D), lambda i: (0, 0)),
        ],
        out_specs=[
            pl.BlockSpec((BN, D), lambda i: (i, 0)),
            pl.BlockSpec((BN, D), lambda i: (i, 0)),
        ],
        out_shape=[
            jax.ShapeDtypeStruct((N, D), jnp.float32),
            jax.ShapeDtypeStruct((N, D), jnp.float32),
        ],
    )(aggp, degp, w, b)


def _dense_body(aggp_ref, degb_ref, w_ref, b_ref, h_ref):
    p = aggp_ref[...]
    x = (p[0] + p[1]) / degb_ref[...]
    y = jnp.dot(x, w_ref[...], preferred_element_type=jnp.float32)
    h_ref[...] = jnp.maximum(y + b_ref[...], 0.0)


def _dense(aggp, degb, w, b):
    return pl.pallas_call(
        _dense_body,
        grid=(N // BN,),
        in_specs=[
            pl.BlockSpec((NC, BN, D), lambda i: (0, i, 0)),
            pl.BlockSpec((BN, D), lambda i: (i, 0)),
            pl.BlockSpec((D, D), lambda i: (0, 0)),
            pl.BlockSpec((1, D), lambda i: (0, 0)),
        ],
        out_specs=pl.BlockSpec((BN, D), lambda i: (i, 0)),
        out_shape=jax.ShapeDtypeStruct((N, D), jnp.float32),
    )(aggp, degb, w, b)


def kernel(h, edge_index, W0, b0, W1, b1, W2, b2):
    src = edge_index[0].reshape(NW, NCH, CH)
    dst = edge_index[1].reshape(NW, NCH, CH)
    degp = _sc_deg(dst)
    aggp = _sc_agg(h, src, dst)
    h1, degb = _dense0(aggp, degp, W0, b0.reshape(1, D))
    aggp = _sc_agg(h1, src, dst)
    h2 = _dense(aggp, degb, W1, b1.reshape(1, D))
    aggp = _sc_agg(h2, src, dst)
    return _dense(aggp, degb, W2, b2.reshape(1, D))
